# baseline (speedup 1.0000x reference)
import jax
import jax.numpy as jnp
from jax import lax
from jax.experimental import pallas as pl
from jax.experimental.pallas import tpu as pltpu
from jax.experimental.pallas import tpu_sc as plsc

_N = 32
_A = 16
_XD = 512
_L = 16
_NW = 32
_RC = 32


def _head_body(x_ref, pmf_ref, wrow_ref, w1_ref, w2_ref, b_ref, avail_ref,
               act_ref, alp_ref):
    x = x_ref[...]
    pmfs = pmf_ref[...].astype(jnp.float32) * wrow_ref[...]
    logits = jnp.dot(x, w1_ref[...], preferred_element_type=jnp.float32)
    logits = logits + jnp.dot(pmfs, w2_ref[...], preferred_element_type=jnp.float32)
    logits = logits + b_ref[...]
    masked = jnp.where(avail_ref[...] > 0, logits, -1e10)
    m = jnp.max(masked, axis=-1, keepdims=True)
    lse = jnp.log(jnp.sum(jnp.exp(masked - m), axis=-1, keepdims=True))
    act_ref[...] = jnp.argmax(masked, axis=-1, keepdims=True).astype(jnp.int32)
    alp_ref[...] = -lse


def _sc_father(pmfT_hbm, w_hbm, father_hbm, fbuf0, fbuf1, pbuf, wv, wbc, colc,
               sem0, sem1):
    T = pmfT_hbm.shape[1]
    kpt = _N * _N // _NW
    cw = kpt * _A
    wid = lax.axis_index("s") * 2 + lax.axis_index("c")
    kbase = wid * kpt
    cbase = wid * cw

    pltpu.sync_copy(w_hbm, wv)
    pltpu.sync_copy(pmfT_hbm.at[pl.ds(kbase, kpt)], pbuf)

    zeros = jnp.zeros((_L,), jnp.float32)

    def zero_body(r, c):
        def zcol(j, cc):
            fbuf0[r, pl.ds(j * _L, _L)] = zeros
            fbuf1[r, pl.ds(j * _L, _L)] = zeros
            return cc

        return lax.fori_loop(0, cw // _L, zcol, c)

    lax.fori_loop(0, _RC, zero_body, 0)

    izero = jnp.zeros((_L,), jnp.int32)
    row_lo = lax.iota(jnp.int32, _L)
    row_hi = row_lo + _L

    def tab_body(kl, _):
        wbc[kl, pl.ds(0, _L)] = plsc.load_gather(wv, [izero + kl])
        colc[kl, pl.ds(0, _L)] = izero + kl * _A
        return 0

    lax.fori_loop(0, kpt, tab_body, 0)

    def do_chunk(c, fbuf):
        i0 = c * _RC

        def k_body(kl, _):
            wb = wbc[kl, pl.ds(0, _L)]
            cols = colc[kl, pl.ds(0, _L)]
            pv0 = pbuf[kl, pl.ds(i0, _L)].astype(jnp.float32)
            plsc.store_scatter(fbuf, [row_lo, cols], pv0 * wb)
            pv1 = pbuf[kl, pl.ds(i0 + _L, _L)].astype(jnp.float32)
            plsc.store_scatter(fbuf, [row_hi, cols], pv1 * wb)
            return 0

        lax.fori_loop(0, kpt, k_body, 0, unroll=8)

    def start(fbuf, i0, sem):
        pltpu.make_async_copy(
            fbuf, father_hbm.at[pl.ds(i0, _RC), pl.ds(cbase, cw)], sem).start()

    def wait(fbuf, i0, sem):
        pltpu.make_async_copy(
            fbuf, father_hbm.at[pl.ds(i0, _RC), pl.ds(cbase, cw)], sem).wait()

    do_chunk(0, fbuf0)
    start(fbuf0, 0, sem0)
    do_chunk(1, fbuf1)
    start(fbuf1, _RC, sem1)

    def step_body(s, _):
        c0 = s * 2
        wait(fbuf0, 0, sem0)
        do_chunk(c0, fbuf0)
        start(fbuf0, c0 * _RC, sem0)
        c1 = c0 + 1
        wait(fbuf1, 0, sem1)
        do_chunk(c1, fbuf1)
        start(fbuf1, c1 * _RC, sem1)
        return 0

    lax.fori_loop(1, T // _RC // 2, step_body, 0)
    wait(fbuf0, 0, sem0)
    wait(fbuf1, 0, sem1)


def kernel(x, parents_mask, available_actions, father_action_weights, W, b,
           deterministic=True):
    T = x.shape[0]
    n = _N
    A = _A
    nn = n * n
    pmf2d = parents_mask.reshape(T, nn)
    pmfT = pmf2d.T
    wvec = jnp.tile(father_action_weights, n)
    W1 = W[:_XD]
    W2 = W[_XD::A]

    row_w = nn * A
    kpt = nn // _NW
    sc_father = pl.kernel(
        _sc_father,
        out_type=jax.ShapeDtypeStruct((T, row_w), jnp.float32),
        mesh=plsc.VectorSubcoreMesh(core_axis_name="c", subcore_axis_name="s"),
        compiler_params=pltpu.CompilerParams(needs_layout_passes=False),
        scratch_types=[
            pltpu.VMEM((_RC, kpt * A), jnp.float32),
            pltpu.VMEM((_RC, kpt * A), jnp.float32),
            pltpu.VMEM((kpt, T), jnp.int32),
            pltpu.VMEM((_N,), jnp.float32),
            pltpu.VMEM((kpt, _L), jnp.float32),
            pltpu.VMEM((kpt, _L), jnp.int32),
            pltpu.SemaphoreType.DMA,
            pltpu.SemaphoreType.DMA,
        ],
    )
    father = sc_father(pmfT, father_action_weights)

    actions, alp = pl.pallas_call(
        _head_body,
        in_specs=[pl.BlockSpec(memory_space=pltpu.VMEM)] * 7,
        out_specs=[pl.BlockSpec(memory_space=pltpu.VMEM)] * 2,
        out_shape=[
            jax.ShapeDtypeStruct((T, 1), jnp.int32),
            jax.ShapeDtypeStruct((T, 1), jnp.float32),
        ],
    )(x, pmf2d, wvec.reshape(1, nn), W1, W2, b.reshape(1, A),
      available_actions)

    return (actions, alp, father)


# speedup vs baseline: 1.0201x; 1.0201x over previous
import jax
import jax.numpy as jnp
from jax import lax
from jax.experimental import pallas as pl
from jax.experimental.pallas import tpu as pltpu
from jax.experimental.pallas import tpu_sc as plsc

_N = 32
_A = 16
_XD = 512
_L = 16
_NW = 32
_RC = 32


def _head_body(x_ref, pmf_ref, wrow_ref, w1_ref, w2_ref, b_ref, avail_ref,
               act_ref, alp_ref):
    x = x_ref[...]
    pmfs = pmf_ref[...].astype(jnp.float32) * wrow_ref[...]
    logits = jnp.dot(x, w1_ref[...], preferred_element_type=jnp.float32)
    logits = logits + jnp.dot(pmfs, w2_ref[...], preferred_element_type=jnp.float32)
    logits = logits + b_ref[...]
    masked = jnp.where(avail_ref[...] > 0, logits, -1e10)
    m = jnp.max(masked, axis=-1, keepdims=True)
    lse = jnp.log(jnp.sum(jnp.exp(masked - m), axis=-1, keepdims=True))
    act_ref[...] = jnp.argmax(masked, axis=-1, keepdims=True).astype(jnp.int32)
    alp_ref[...] = -lse


def _sc_father(pmfT_hbm, w_hbm, father_hbm, fbuf0, fbuf1, pbuf, wv, wbc, colc,
               sem0, sem1):
    T = pmfT_hbm.shape[1]
    kpt = _N * _N // _NW
    cw = kpt * _A
    wid = lax.axis_index("s") * 2 + lax.axis_index("c")
    kbase = wid * kpt
    cbase = wid * cw

    pltpu.sync_copy(w_hbm, wv)
    pltpu.sync_copy(pmfT_hbm.at[pl.ds(kbase, kpt)], pbuf)

    zeros = jnp.zeros((_L,), jnp.float32)

    def zero_body(r, c):
        def zcol(j, cc):
            fbuf0[r, pl.ds(j * _L, _L)] = zeros
            fbuf1[r, pl.ds(j * _L, _L)] = zeros
            return cc

        return lax.fori_loop(0, cw // _L, zcol, c)

    lax.fori_loop(0, _RC, zero_body, 0)

    izero = jnp.zeros((_L,), jnp.int32)
    row_lo = lax.iota(jnp.int32, _L)
    row_hi = row_lo + _L

    def tab_body(kl, _):
        wbc[kl, pl.ds(0, _L)] = plsc.load_gather(wv, [izero + kl])
        colc[kl, pl.ds(0, _L)] = izero + kl * _A
        return 0

    lax.fori_loop(0, kpt, tab_body, 0)

    def do_chunk(c, fbuf):
        i0 = c * _RC

        def k_body(kl, _):
            wb = wbc[kl, pl.ds(0, _L)]
            cols = colc[kl, pl.ds(0, _L)]
            pv0 = pbuf[kl, pl.ds(i0, _L)].astype(jnp.float32)
            plsc.store_scatter(fbuf, [row_lo, cols], pv0 * wb)
            pv1 = pbuf[kl, pl.ds(i0 + _L, _L)].astype(jnp.float32)
            plsc.store_scatter(fbuf, [row_hi, cols], pv1 * wb)
            return 0

        lax.fori_loop(0, kpt, k_body, 0, unroll=4)

    def start(fbuf, i0, sem):
        pltpu.make_async_copy(
            fbuf, father_hbm.at[pl.ds(i0, _RC), pl.ds(cbase, cw)], sem).start()

    def wait(fbuf, i0, sem):
        pltpu.make_async_copy(
            fbuf, father_hbm.at[pl.ds(i0, _RC), pl.ds(cbase, cw)], sem).wait()

    do_chunk(0, fbuf0)
    start(fbuf0, 0, sem0)
    do_chunk(1, fbuf1)
    start(fbuf1, _RC, sem1)

    def step_body(s, _):
        c0 = s * 2
        wait(fbuf0, 0, sem0)
        do_chunk(c0, fbuf0)
        start(fbuf0, c0 * _RC, sem0)
        c1 = c0 + 1
        wait(fbuf1, 0, sem1)
        do_chunk(c1, fbuf1)
        start(fbuf1, c1 * _RC, sem1)
        return 0

    lax.fori_loop(1, T // _RC // 2, step_body, 0)
    wait(fbuf0, 0, sem0)
    wait(fbuf1, 0, sem1)


def kernel(x, parents_mask, available_actions, father_action_weights, W, b,
           deterministic=True):
    T = x.shape[0]
    n = _N
    A = _A
    nn = n * n
    pmf2d = parents_mask.reshape(T, nn)
    pmfT = pmf2d.T
    wvec = jnp.tile(father_action_weights, n)
    W1 = W[:_XD]
    W2 = W[_XD::A]

    row_w = nn * A
    kpt = nn // _NW
    sc_father = pl.kernel(
        _sc_father,
        out_type=jax.ShapeDtypeStruct((T, row_w), jnp.float32),
        mesh=plsc.VectorSubcoreMesh(core_axis_name="c", subcore_axis_name="s"),
        compiler_params=pltpu.CompilerParams(needs_layout_passes=False),
        scratch_types=[
            pltpu.VMEM((_RC, kpt * A), jnp.float32),
            pltpu.VMEM((_RC, kpt * A), jnp.float32),
            pltpu.VMEM((kpt, T), jnp.int32),
            pltpu.VMEM((_N,), jnp.float32),
            pltpu.VMEM((kpt, _L), jnp.float32),
            pltpu.VMEM((kpt, _L), jnp.int32),
            pltpu.SemaphoreType.DMA,
            pltpu.SemaphoreType.DMA,
        ],
    )
    father = sc_father(pmfT, father_action_weights)

    actions, alp = pl.pallas_call(
        _head_body,
        in_specs=[pl.BlockSpec(memory_space=pltpu.VMEM)] * 7,
        out_specs=[pl.BlockSpec(memory_space=pltpu.VMEM)] * 2,
        out_shape=[
            jax.ShapeDtypeStruct((T, 1), jnp.int32),
            jax.ShapeDtypeStruct((T, 1), jnp.float32),
        ],
    )(x, pmf2d, wvec.reshape(1, nn), W1, W2, b.reshape(1, A),
      available_actions)

    return (actions, alp, father)


# speedup vs baseline: 1.0393x; 1.0188x over previous
import jax
import jax.numpy as jnp
from jax import lax
from jax.experimental import pallas as pl
from jax.experimental.pallas import tpu as pltpu
from jax.experimental.pallas import tpu_sc as plsc

_N = 32
_A = 16
_XD = 512
_L = 16
_NW = 32
_RC = 32


def _head_body(x_ref, pmf_ref, wrow_ref, w1_ref, w2_ref, b_ref, avail_ref,
               act_ref, alp_ref):
    x = x_ref[...]
    pmfs = pmf_ref[...].astype(jnp.float32) * wrow_ref[...]
    logits = jnp.dot(x, w1_ref[...], preferred_element_type=jnp.float32)
    logits = logits + jnp.dot(pmfs, w2_ref[...], preferred_element_type=jnp.float32)
    logits = logits + b_ref[...]
    masked = jnp.where(avail_ref[...] > 0, logits, -1e10)
    m = jnp.max(masked, axis=-1, keepdims=True)
    lse = jnp.log(jnp.sum(jnp.exp(masked - m), axis=-1, keepdims=True))
    act_ref[...] = jnp.argmax(masked, axis=-1, keepdims=True).astype(jnp.int32)
    alp_ref[...] = -lse


def _sc_father(pmfT_hbm, w_hbm, father_hbm, fbuf0, fbuf1, pbuf, wv, sem0, sem1):
    T = pmfT_hbm.shape[1]
    kpt = _N * _N // _NW
    cw = kpt * _A
    wid = lax.axis_index("s") * 2 + lax.axis_index("c")
    kbase = wid * kpt
    cbase = wid * cw

    pltpu.sync_copy(w_hbm, wv)
    pltpu.sync_copy(pmfT_hbm.at[pl.ds(kbase, kpt)], pbuf)

    zeros = jnp.zeros((_L,), jnp.float32)

    def zero_body(r, c):
        def zcol(j, cc):
            fbuf0[r, pl.ds(j * _L, _L)] = zeros
            fbuf1[r, pl.ds(j * _L, _L)] = zeros
            return cc

        return lax.fori_loop(0, cw // _L, zcol, c)

    lax.fori_loop(0, _RC, zero_body, 0)

    izero = jnp.zeros((_L,), jnp.int32)
    row_lo = lax.iota(jnp.int32, _L)
    row_hi = row_lo + _L

    def do_chunk(c, fbuf):
        i0 = c * _RC

        def k_body(kl, _):
            wb = plsc.load_gather(wv, [izero + kl])
            cols = izero + kl * _A
            pv0 = pbuf[kl, pl.ds(i0, _L)].astype(jnp.float32)
            plsc.store_scatter(fbuf, [row_lo, cols], pv0 * wb)
            pv1 = pbuf[kl, pl.ds(i0 + _L, _L)].astype(jnp.float32)
            plsc.store_scatter(fbuf, [row_hi, cols], pv1 * wb)
            return 0

        lax.fori_loop(0, kpt, k_body, 0, unroll=2)

    def start(fbuf, i0, sem):
        pltpu.make_async_copy(
            fbuf, father_hbm.at[pl.ds(i0, _RC), pl.ds(cbase, cw)], sem).start()

    def wait(fbuf, i0, sem):
        pltpu.make_async_copy(
            fbuf, father_hbm.at[pl.ds(i0, _RC), pl.ds(cbase, cw)], sem).wait()

    start(fbuf0, 0, sem0)
    start(fbuf1, _RC, sem1)

    def step_body(s, _):
        c0 = s * 2
        wait(fbuf0, 0, sem0)
        do_chunk(c0, fbuf0)
        start(fbuf0, c0 * _RC, sem0)
        c1 = c0 + 1
        wait(fbuf1, 0, sem1)
        do_chunk(c1, fbuf1)
        start(fbuf1, c1 * _RC, sem1)
        return 0

    lax.fori_loop(0, T // _RC // 2, step_body, 0)
    wait(fbuf0, 0, sem0)
    wait(fbuf1, 0, sem1)


def kernel(x, parents_mask, available_actions, father_action_weights, W, b,
           deterministic=True):
    T = x.shape[0]
    n = _N
    A = _A
    nn = n * n
    pmf2d = parents_mask.reshape(T, nn)
    pmfT = pmf2d.T
    wvec = jnp.tile(father_action_weights, n)
    W1 = W[:_XD]
    W2 = W[_XD::A]

    row_w = nn * A
    kpt = nn // _NW
    sc_father = pl.kernel(
        _sc_father,
        out_type=jax.ShapeDtypeStruct((T, row_w), jnp.float32),
        mesh=plsc.VectorSubcoreMesh(core_axis_name="c", subcore_axis_name="s"),
        compiler_params=pltpu.CompilerParams(needs_layout_passes=False),
        scratch_types=[
            pltpu.VMEM((_RC, kpt * A), jnp.float32),
            pltpu.VMEM((_RC, kpt * A), jnp.float32),
            pltpu.VMEM((kpt, T), jnp.int32),
            pltpu.VMEM((_N,), jnp.float32),
            pltpu.SemaphoreType.DMA,
            pltpu.SemaphoreType.DMA,
        ],
    )
    father = sc_father(pmfT, father_action_weights)

    actions, alp = pl.pallas_call(
        _head_body,
        in_specs=[pl.BlockSpec(memory_space=pltpu.VMEM)] * 7,
        out_specs=[pl.BlockSpec(memory_space=pltpu.VMEM)] * 2,
        out_shape=[
            jax.ShapeDtypeStruct((T, 1), jnp.int32),
            jax.ShapeDtypeStruct((T, 1), jnp.float32),
        ],
    )(x, pmf2d, wvec.reshape(1, nn), W1, W2, b.reshape(1, A),
      available_actions)

    return (actions, alp, father)


# speedup vs baseline: 1.0608x; 1.0207x over previous
import jax
import jax.numpy as jnp
from jax import lax
from jax.experimental import pallas as pl
from jax.experimental.pallas import tpu as pltpu
from jax.experimental.pallas import tpu_sc as plsc

_N = 32
_A = 16
_XD = 512
_L = 16
_NW = 32


def _head_body(x_ref, pmf_ref, wrow_ref, w1_ref, w2_ref, b_ref, avail_ref,
               act_ref, alp_ref):
    x = x_ref[...]
    pmfs = pmf_ref[...].astype(jnp.float32) * wrow_ref[...]
    logits = jnp.dot(x, w1_ref[...], preferred_element_type=jnp.float32)
    logits = logits + jnp.dot(pmfs, w2_ref[...], preferred_element_type=jnp.float32)
    logits = logits + b_ref[...]
    masked = jnp.where(avail_ref[...] > 0, logits, -1e10)
    m = jnp.max(masked, axis=-1, keepdims=True)
    lse = jnp.log(jnp.sum(jnp.exp(masked - m), axis=-1, keepdims=True))
    act_ref[...] = jnp.argmax(masked, axis=-1, keepdims=True).astype(jnp.int32)
    alp_ref[...] = -lse


def _sc_father(pmf_hbm, w_hbm, father_hbm,
               fbuf0, fbuf1, pbuf, wv, sem0, sem1):
    nn = _N * _N
    row_w = nn * _A
    wid = lax.axis_index("s") * 2 + lax.axis_index("c")
    rows = pmf_hbm.shape[0] // _NW
    base = wid * rows

    pltpu.sync_copy(w_hbm, wv)
    pltpu.sync_copy(pmf_hbm.at[pl.ds(base, rows)], pbuf)

    zeros = jnp.zeros((_L,), jnp.float32)

    def zero_body(i, c):
        fbuf0[pl.ds(i * _L, _L)] = zeros
        fbuf1[pl.ds(i * _L, _L)] = zeros
        return c

    lax.fori_loop(0, row_w // _L, zero_body, 0)

    lane_off = lax.iota(jnp.int32, _L) * _A
    w_lo = wv[pl.ds(0, _L)]
    w_hi = wv[pl.ds(_L, _L)]

    def do_row(rr, fbuf):
        def g_body(h, _):
            g0 = h * 2
            pv0 = pbuf[rr, pl.ds(g0 * _L, _L)].astype(jnp.float32)
            plsc.store_scatter(fbuf, [lane_off + g0 * (_L * _A)], pv0 * w_lo)
            g1 = g0 + 1
            pv1 = pbuf[rr, pl.ds(g1 * _L, _L)].astype(jnp.float32)
            plsc.store_scatter(fbuf, [lane_off + g1 * (_L * _A)], pv1 * w_hi)
            return 0

        lax.fori_loop(0, nn // (2 * _L), g_body, 0, unroll=4)

    def start(fbuf, r, sem):
        pltpu.make_async_copy(fbuf, father_hbm.at[r], sem).start()

    def wait(fbuf, r, sem):
        pltpu.make_async_copy(fbuf, father_hbm.at[r], sem).wait()

    start(fbuf0, base + 0, sem0)
    start(fbuf1, base + 1, sem1)

    def step_body(s, _):
        rr0 = s * 2
        wait(fbuf0, base + rr0, sem0)
        do_row(rr0, fbuf0)
        start(fbuf0, base + rr0, sem0)
        rr1 = rr0 + 1
        wait(fbuf1, base + rr1, sem1)
        do_row(rr1, fbuf1)
        start(fbuf1, base + rr1, sem1)
        return 0

    lax.fori_loop(0, rows // 2, step_body, 0)
    wait(fbuf0, base, sem0)
    wait(fbuf1, base, sem1)


def kernel(x, parents_mask, available_actions, father_action_weights, W, b,
           deterministic=True):
    T = x.shape[0]
    n = _N
    A = _A
    nn = n * n
    pmf2d = parents_mask.reshape(T, nn)
    wvec = jnp.tile(father_action_weights, n)
    W1 = W[:_XD]
    W2 = W[_XD::A]

    row_w = nn * A
    rows = T // _NW
    sc_father = pl.kernel(
        _sc_father,
        out_type=jax.ShapeDtypeStruct((T, row_w), jnp.float32),
        mesh=plsc.VectorSubcoreMesh(core_axis_name="c", subcore_axis_name="s"),
        compiler_params=pltpu.CompilerParams(needs_layout_passes=False),
        scratch_types=[
            pltpu.VMEM((row_w,), jnp.float32),
            pltpu.VMEM((row_w,), jnp.float32),
            pltpu.VMEM((rows, nn), jnp.int32),
            pltpu.VMEM((_N,), jnp.float32),
            pltpu.SemaphoreType.DMA,
            pltpu.SemaphoreType.DMA,
        ],
    )
    father = sc_father(pmf2d, father_action_weights)

    actions, alp = pl.pallas_call(
        _head_body,
        in_specs=[pl.BlockSpec(memory_space=pltpu.VMEM)] * 7,
        out_specs=[pl.BlockSpec(memory_space=pltpu.VMEM)] * 2,
        out_shape=[
            jax.ShapeDtypeStruct((T, 1), jnp.int32),
            jax.ShapeDtypeStruct((T, 1), jnp.float32),
        ],
    )(x, pmf2d, wvec.reshape(1, nn), W1, W2, b.reshape(1, A),
      available_actions)

    return (actions, alp, father)


# speedup vs baseline: 1.0852x; 1.0230x over previous
import jax
import jax.numpy as jnp
from jax import lax
from jax.experimental import pallas as pl
from jax.experimental.pallas import tpu as pltpu
from jax.experimental.pallas import tpu_sc as plsc

_N = 32
_A = 16
_XD = 512
_L = 16
_NW = 32


def _head_body(x_ref, pmf_ref, wrow_ref, w1_ref, w2_ref, b_ref, avail_ref,
               act_ref, alp_ref):
    x = x_ref[...]
    pmfs = pmf_ref[...].astype(jnp.float32) * wrow_ref[...]
    logits = jnp.dot(x, w1_ref[...], preferred_element_type=jnp.float32)
    logits = logits + jnp.dot(pmfs, w2_ref[...], preferred_element_type=jnp.float32)
    logits = logits + b_ref[...]
    masked = jnp.where(avail_ref[...] > 0, logits, -1e10)
    m = jnp.max(masked, axis=-1, keepdims=True)
    lse = jnp.log(jnp.sum(jnp.exp(masked - m), axis=-1, keepdims=True))
    act_ref[...] = jnp.argmax(masked, axis=-1, keepdims=True).astype(jnp.int32)
    alp_ref[...] = -lse


def _sc_father(pmf_hbm, w_hbm, father_hbm,
               fbuf0, fbuf1, pbuf, wv, sem0, sem1):
    nn = _N * _N
    row_w = nn * _A
    wid = lax.axis_index("s") * 2 + lax.axis_index("c")
    rows = pmf_hbm.shape[0] // _NW
    base = wid * rows

    pltpu.sync_copy(w_hbm, wv)
    pltpu.sync_copy(pmf_hbm.at[pl.ds(base, rows)], pbuf)

    zeros = jnp.zeros((_L,), jnp.float32)

    def zero_body(i, c):
        fbuf0[pl.ds(i * _L, _L)] = zeros
        fbuf1[pl.ds(i * _L, _L)] = zeros
        return c

    lax.fori_loop(0, row_w // _L, zero_body, 0)

    lane_off = lax.iota(jnp.int32, _L) * _A
    w_lo = wv[pl.ds(0, _L)]
    w_hi = wv[pl.ds(_L, _L)]

    def do_row(rr, fbuf):
        def g_body(h, _):
            g0 = h * 2
            pv0 = pbuf[rr, pl.ds(g0 * _L, _L)].astype(jnp.float32)
            plsc.store_scatter(fbuf, [lane_off + g0 * (_L * _A)], pv0 * w_lo)
            g1 = g0 + 1
            pv1 = pbuf[rr, pl.ds(g1 * _L, _L)].astype(jnp.float32)
            plsc.store_scatter(fbuf, [lane_off + g1 * (_L * _A)], pv1 * w_hi)
            return 0

        lax.fori_loop(0, nn // (2 * _L), g_body, 0, unroll=4)

    def start(fbuf, r, sem):
        pltpu.make_async_copy(fbuf, father_hbm.at[r], sem).start()

    def wait(fbuf, r, sem):
        pltpu.make_async_copy(fbuf, father_hbm.at[r], sem).wait()

    do_row(0, fbuf0)
    start(fbuf0, base + 0, sem0)
    do_row(1, fbuf1)
    start(fbuf1, base + 1, sem1)

    def step_body(s, _):
        rr0 = s * 2
        wait(fbuf0, base + rr0, sem0)
        do_row(rr0, fbuf0)
        start(fbuf0, base + rr0, sem0)
        rr1 = rr0 + 1
        wait(fbuf1, base + rr1, sem1)
        do_row(rr1, fbuf1)
        start(fbuf1, base + rr1, sem1)
        return 0

    lax.fori_loop(1, rows // 2, step_body, 0)
    wait(fbuf0, base, sem0)
    wait(fbuf1, base, sem1)


def kernel(x, parents_mask, available_actions, father_action_weights, W, b,
           deterministic=True):
    T = x.shape[0]
    n = _N
    A = _A
    nn = n * n
    pmf2d = parents_mask.reshape(T, nn)
    wvec = jnp.tile(father_action_weights, n)
    W1 = W[:_XD]
    W2 = W[_XD::A]

    row_w = nn * A
    rows = T // _NW
    sc_father = pl.kernel(
        _sc_father,
        out_type=jax.ShapeDtypeStruct((T, row_w), jnp.float32),
        mesh=plsc.VectorSubcoreMesh(core_axis_name="c", subcore_axis_name="s"),
        compiler_params=pltpu.CompilerParams(needs_layout_passes=False),
        scratch_types=[
            pltpu.VMEM((row_w,), jnp.float32),
            pltpu.VMEM((row_w,), jnp.float32),
            pltpu.VMEM((rows, nn), jnp.int32),
            pltpu.VMEM((_N,), jnp.float32),
            pltpu.SemaphoreType.DMA,
            pltpu.SemaphoreType.DMA,
        ],
    )
    father = sc_father(pmf2d, father_action_weights)

    actions, alp = pl.pallas_call(
        _head_body,
        in_specs=[pl.BlockSpec(memory_space=pltpu.VMEM)] * 7,
        out_specs=[pl.BlockSpec(memory_space=pltpu.VMEM)] * 2,
        out_shape=[
            jax.ShapeDtypeStruct((T, 1), jnp.int32),
            jax.ShapeDtypeStruct((T, 1), jnp.float32),
        ],
    )(x, pmf2d, wvec.reshape(1, nn), W1, W2, b.reshape(1, A),
      available_actions)

    return (actions, alp, father)
